# baseline (speedup 1.0000x reference)
import functools

import jax
import jax.numpy as jnp
from jax import lax
from jax.experimental import pallas as pl
from jax.experimental.pallas import tpu as pltpu
from jax.experimental.pallas import tpu_sc as plsc

L = 16
U = 8
PAD = U * L
FULL_PASSES = 6
MID_PASSES = 5
TAIL_PASSES = 19

_DNUMS = lax.GatherDimensionNumbers(
    offset_dims=(), collapsed_slice_dims=(0,), start_index_map=(0,))


def _shuffle(v, idx):
    return lax.gather(v, idx[:, None], _DNUMS, (1,),
                      mode=lax.GatherScatterMode.PROMISE_IN_BOUNDS)


def _butterfly(v, op):
    lane = lax.iota(jnp.int32, L)
    for s in (8, 4, 2, 1):
        v = op(v, _shuffle(v, jnp.bitwise_xor(lane, s)))
    return v


def _dyn_loop(nvr, body, carry):
    trips = (nvr + U - 1) // U

    def gb(g, c):
        for u in range(U):
            c = body(g * U + u, c)
        return c

    return lax.fori_loop(0, trips, gb, carry)


def _count_pass(buf, nvr, mid, dynamic):
    def cbody(i, cnt):
        v = buf[pl.ds(i * L, L)]
        return cnt + jnp.where(v > mid, 1.0, 0.0)

    z = jnp.zeros((L,), jnp.float32)
    if dynamic:
        cnt_v = _dyn_loop(nvr, cbody, z)
    else:
        cnt_v = lax.fori_loop(0, nvr, cbody, z, unroll=U)
    return _butterfly(cnt_v, jnp.add)[0]


def _bisect(buf, nvr, lo, hi, kf, c_disc, n, dynamic):
    def pb(_, carry):
        lo, hi = carry
        mid = 0.5 * lo + 0.5 * hi
        c = c_disc + _count_pass(buf, nvr, mid, dynamic)
        ge = c >= kf
        return jnp.where(ge, mid, lo), jnp.where(ge, hi, mid)

    return lax.fori_loop(0, n, pb, (lo, hi))


def _compact(buf, nvr, lo, hi, c_disc, s_disc, dynamic):
    nan_v = jnp.full((L,), jnp.nan, jnp.float32)

    def cb(i, carry):
        out, cd, sd = carry
        v = buf[pl.ds(i * L, L)]
        ib = jnp.where((v > lo) & (v <= hi), 1.0, 0.0)
        keep = _butterfly(ib, jnp.maximum)[0] > 0.0
        gt = v > hi
        cnt = jnp.where(gt, 1.0, 0.0)
        sv = jnp.where(gt, v, 0.0)

        @pl.when(keep)
        def _():
            buf[pl.ds(out * L, L)] = v

        cd = cd + jnp.where(keep, 0.0, _butterfly(cnt, jnp.add)[0])
        sd = sd + jnp.where(keep, 0.0, _butterfly(sv, jnp.add)[0])
        return jnp.where(keep, out + 1, out), cd, sd

    z = jnp.float32(0.0)
    if dynamic:
        out, cd, sd = _dyn_loop(nvr, cb, (jnp.int32(0), z, z))
    else:
        out, cd, sd = lax.fori_loop(0, nvr, cb, (jnp.int32(0), z, z))
    for u in range(U):
        buf[pl.ds((out + u) * L, L)] = nan_v
    return out, c_disc + cd, s_disc + sd


def _row_topk_sum(buf, n, k):
    nvr0 = n // L
    kf = jnp.float32(k)

    def mm_body(i, carry):
        vmin, vmax = carry
        v = buf[pl.ds(i * L, L)]
        return jnp.minimum(vmin, v), jnp.maximum(vmax, v)

    vmin, vmax = lax.fori_loop(
        0, nvr0, mm_body,
        (jnp.full((L,), jnp.inf, jnp.float32),
         jnp.full((L,), -jnp.inf, jnp.float32)),
        unroll=U)
    lo = -_butterfly(-vmin, jnp.maximum)[0]
    hi = _butterfly(vmax, jnp.maximum)[0]

    zero = jnp.float32(0.0)
    lo, hi = _bisect(buf, nvr0, lo, hi, kf, zero, FULL_PASSES, dynamic=False)
    nvr1, c_disc, s_disc = _compact(buf, nvr0, lo, hi, zero, zero,
                                    dynamic=False)
    lo, hi = _bisect(buf, nvr1, lo, hi, kf, c_disc, MID_PASSES, dynamic=True)
    nvr2, c_disc, s_disc = _compact(buf, nvr1, lo, hi, c_disc, s_disc,
                                    dynamic=True)
    lo, hi = _bisect(buf, nvr2, lo, hi, kf, c_disc, TAIL_PASSES, dynamic=True)

    def fbody(i, carry):
        cnt, sv = carry
        v = buf[pl.ds(i * L, L)]
        m = v > hi
        return cnt + jnp.where(m, 1.0, 0.0), sv + jnp.where(m, v, 0.0)

    cnt_v, sum_v = _dyn_loop(
        nvr2, fbody,
        (jnp.zeros((L,), jnp.float32), jnp.zeros((L,), jnp.float32)))
    c = c_disc + _butterfly(cnt_v, jnp.add)[0]
    s = s_disc + _butterfly(sum_v, jnp.add)[0]
    return s + (kf - c) * hi


def _sc_kernel(rows, cols, k, rows_per_w):
    nc = 2
    mesh = plsc.VectorSubcoreMesh(core_axis_name="c", subcore_axis_name="s")

    @functools.partial(
        pl.kernel,
        out_type=jax.ShapeDtypeStruct((rows // rows_per_w, L), jnp.float32),
        mesh=mesh,
        scratch_types=[
            pltpu.VMEM((cols + PAD,), jnp.float32),
            pltpu.VMEM((L,), jnp.float32),
        ],
    )
    def run(loss_hbm, out_hbm, buf, out_v):
        wid = lax.axis_index("s") * nc + lax.axis_index("c")
        lane = lax.iota(jnp.int32, L)
        total = jnp.float32(0.0)
        for r in range(rows_per_w):
            row = wid * rows_per_w + r
            pltpu.sync_copy(loss_hbm.at[row], buf.at[pl.ds(0, cols)])
            total = total + _row_topk_sum(buf, cols, k)
        out_v[...] = jnp.where(lane == 0, total, 0.0)
        pltpu.sync_copy(out_v, out_hbm.at[wid])

    return run


def kernel(loss):
    b = loss.shape[0]
    loss2 = loss.reshape(b, -1)
    p = loss2.shape[1]
    k = int(0.25 * p)
    nw = 32
    rows_per_w = b // nw
    partials = _sc_kernel(b, p, k, rows_per_w)(loss2)
    return jnp.sum(partials) / jnp.float32(b * k)


# speedup vs baseline: 2.0735x; 2.0735x over previous
import functools

import jax
import jax.numpy as jnp
from jax import lax
from jax.experimental import pallas as pl
from jax.experimental.pallas import tpu as pltpu
from jax.experimental.pallas import tpu_sc as plsc

L = 16
U = 8
PASSES = 24

_DNUMS = lax.GatherDimensionNumbers(
    offset_dims=(), collapsed_slice_dims=(0,), start_index_map=(0,))


def _shuffle(v, idx):
    return lax.gather(v, idx[:, None], _DNUMS, (1,),
                      mode=lax.GatherScatterMode.PROMISE_IN_BOUNDS)


def _butterfly(v, op):
    lane = lax.iota(jnp.int32, L)
    for s in (8, 4, 2, 1):
        v = op(v, _shuffle(v, jnp.bitwise_xor(lane, s)))
    return v


def _group_loop(nvr, body, carry):
    def gb(g, c):
        for u in range(U):
            c = body(g * U + u, u, c)
        return c

    return lax.fori_loop(0, nvr // U, gb, carry)


def _row_topk_sum(buf, n, k):
    nvr = n // L
    kf = jnp.float32(k)
    zero_v = jnp.zeros((L,), jnp.float32)

    def mm_body(i, u, carry):
        a = carry[u % 4]
        v = buf[pl.ds(i * L, L)]
        lohi = (jnp.minimum(a[0], v), jnp.maximum(a[1], v))
        return tuple(lohi if j == u % 4 else carry[j] for j in range(4))

    mm = _group_loop(
        nvr, mm_body,
        tuple((jnp.full((L,), jnp.inf, jnp.float32),
               jnp.full((L,), -jnp.inf, jnp.float32)) for _ in range(4)))
    vmin = jnp.minimum(jnp.minimum(mm[0][0], mm[1][0]),
                       jnp.minimum(mm[2][0], mm[3][0]))
    vmax = jnp.maximum(jnp.maximum(mm[0][1], mm[1][1]),
                       jnp.maximum(mm[2][1], mm[3][1]))
    lo = -_butterfly(-vmin, jnp.maximum)[0]
    hi = _butterfly(vmax, jnp.maximum)[0]

    def count_pass(mid):
        def cbody(i, u, carry):
            v = buf[pl.ds(i * L, L)]
            t = jnp.where(v > mid, 1.0, 0.0)
            return tuple(carry[j] + t if j == u % 4 else carry[j]
                         for j in range(4))

        c = _group_loop(nvr, cbody, (zero_v, zero_v, zero_v, zero_v))
        return _butterfly((c[0] + c[1]) + (c[2] + c[3]), jnp.add)[0]

    def pb(_, carry):
        lo, hi = carry
        mid = 0.5 * lo + 0.5 * hi
        ge = count_pass(mid) >= kf
        return jnp.where(ge, mid, lo), jnp.where(ge, hi, mid)

    lo, hi = lax.fori_loop(0, PASSES, pb, (lo, hi))

    def fbody(i, u, carry):
        v = buf[pl.ds(i * L, L)]
        m = v > hi
        t = (carry[u % 4][0] + jnp.where(m, 1.0, 0.0),
             carry[u % 4][1] + jnp.where(m, v, 0.0))
        return tuple(t if j == u % 4 else carry[j] for j in range(4))

    f = _group_loop(nvr, fbody, tuple((zero_v, zero_v) for _ in range(4)))
    c = _butterfly((f[0][0] + f[1][0]) + (f[2][0] + f[3][0]), jnp.add)[0]
    s = _butterfly((f[0][1] + f[1][1]) + (f[2][1] + f[3][1]), jnp.add)[0]
    return s + (kf - c) * hi


def _sc_kernel(rows, cols, k, rows_per_w):
    nc = 2
    mesh = plsc.VectorSubcoreMesh(core_axis_name="c", subcore_axis_name="s")

    @functools.partial(
        pl.kernel,
        out_type=jax.ShapeDtypeStruct((rows // rows_per_w, L), jnp.float32),
        mesh=mesh,
        scratch_types=[
            pltpu.VMEM((cols,), jnp.float32),
            pltpu.VMEM((cols,), jnp.float32),
            pltpu.VMEM((L,), jnp.float32),
            pltpu.SemaphoreType.DMA,
            pltpu.SemaphoreType.DMA,
        ],
    )
    def run(loss_hbm, out_hbm, buf_a, buf_b, out_v, sem_a, sem_b):
        wid = lax.axis_index("s") * nc + lax.axis_index("c")
        lane = lax.iota(jnp.int32, L)
        bufs = (buf_a, buf_b)
        sems = (sem_a, sem_b)
        base = wid * rows_per_w
        copies = [None] * rows_per_w
        copies[0] = pltpu.async_copy(loss_hbm.at[base], buf_a, sem_a)
        total = jnp.float32(0.0)
        for r in range(rows_per_w):
            if r + 1 < rows_per_w:
                copies[r + 1] = pltpu.async_copy(
                    loss_hbm.at[base + r + 1], bufs[(r + 1) % 2],
                    sems[(r + 1) % 2])
            copies[r].wait()
            total = total + _row_topk_sum(bufs[r % 2], cols, k)
        out_v[...] = jnp.where(lane == 0, total, 0.0)
        pltpu.sync_copy(out_v, out_hbm.at[wid])

    return run


def kernel(loss):
    b = loss.shape[0]
    loss2 = loss.reshape(b, -1)
    p = loss2.shape[1]
    k = int(0.25 * p)
    nw = 32
    rows_per_w = b // nw
    partials = _sc_kernel(b, p, k, rows_per_w)(loss2)
    return jnp.sum(partials) / jnp.float32(b * k)
